# baseline (speedup 1.0000x reference)
import jax
import jax.numpy as jnp
from jax import lax
from jax.experimental import pallas as pl
from jax.experimental.pallas import tpu as pltpu
from jax.experimental.pallas import tpu_sc as plsc

N = 8192
DP = 1024
CUT = (0, 20000, 40000, 80000, 100000)
DS = (1024, 256, 64, 16)
DSP = (1024, 256, 64, 16)
SCALE = float(DP) ** 0.5
TILE = 512
SUB = 64
NT = 20
CPT = 8
CAP = 8192
NW = 32
MAXG = 4
MAXS = 5


def _mesh():
    return plsc.VectorSubcoreMesh(core_axis_name="c", subcore_axis_name="s")


def _plan(inp_flat):
    i32 = jnp.int32
    cid = ((inp_flat >= CUT[1]).astype(i32)
           + (inp_flat >= CUT[2]).astype(i32)
           + (inp_flat >= CUT[3]).astype(i32))
    perm = jnp.argsort(cid).astype(i32)
    n = jnp.stack([jnp.sum(cid == c) for c in range(4)]).astype(i32)
    b = jnp.concatenate([jnp.zeros((1,), i32), jnp.cumsum(n)[:-1].astype(i32)])
    t = (n + TILE - 1) // TILE
    T = jnp.concatenate([jnp.zeros((1,), i32), jnp.cumsum(t)[:-1].astype(i32)])
    ntt = jnp.sum(t)

    jj = jnp.arange(NT, dtype=i32)
    cid_t = ((jj >= T[1]).astype(i32) + (jj >= T[2]).astype(i32)
             + (jj >= T[3]).astype(i32))
    ks = [jnp.clip(jj - T[c], 0, jnp.maximum(t[c] - 1, 0)) for c in range(4)]
    meta = jnp.stack([cid_t] + ks).astype(i32)

    counts16 = jnp.concatenate(
        [t * CPT, (ntt * CPT)[None], jnp.zeros((3,), i32), T * CPT,
         jnp.zeros((4,), i32)]).astype(i32)

    r = jnp.arange(NT * TILE, dtype=i32)
    jr = r // TILE
    lr = r % TILE
    cid_r = ((jr >= T[1]).astype(i32) + (jr >= T[2]).astype(i32)
             + (jr >= T[3]).astype(i32))
    def sel(vals):
        x = jnp.where(cid_r == 1, vals[1], vals[0])
        x = jnp.where(cid_r == 2, vals[2], x)
        return jnp.where(cid_r == 3, vals[3], x)
    off = (jr - sel(T)) * TILE + lr
    valid = (jr < ntt) & (off < sel(n))
    pos = sel(b) + off
    dval = jnp.take(perm, jnp.clip(pos, 0, N - 1))
    tok = jnp.take(inp_flat, dval)
    cut_r = sel([jnp.full((), CUT[c], i32) for c in range(4)])
    gidx = jnp.where(valid, tok - cut_r, 0).astype(i32)
    dest = jnp.where(valid, dval, N).astype(i32)
    slot = jnp.zeros((N + 1,), i32).at[dest].set(r)[:N]
    return meta, gidx, counts16, slot


def _gather_body(t0, t1, t2, t3, g, cnts,
                 b0, b1, b2, b3, idx_v, r0, r1, r2, r3, cnt_v, sem):
    wid = lax.axis_index("s") * 2 + lax.axis_index("c")
    pltpu.sync_copy(cnts, cnt_v)
    cv = cnt_v[...]
    tabs = (t0, t1, t2, t3)
    bufs = (b0, b1, b2, b3)
    rvs = (r0, r1, r2, r3)
    for c in range(4):
        n_ch = cv[c]
        g_off = cv[8 + c]
        for s in range(MAXG):
            ch = s * NW + wid
            @pl.when(ch < n_ch)
            def _(c=c, ch=ch, g_off=g_off):
                pltpu.sync_copy(g.at[pl.ds((g_off + ch) * SUB, SUB)], idx_v)
                if c < 2:
                    pltpu.async_copy(tabs[c].at[idx_v], rvs[c], sem).wait()
                else:
                    descs = []
                    for k in range(SUB // 16):
                        xv = idx_v[pl.ds(k * 16, 16)]
                        for j in range(16):
                            descs.append(pltpu.async_copy(
                                tabs[c].at[xv[j]], rvs[c].at[k * 16 + j], sem))
                    for d in descs:
                        d.wait()
                pltpu.sync_copy(rvs[c], bufs[c].at[pl.ds(ch * SUB, SUB)])


def _unsort_body(src, sidx, out, rows_v, sidx_v, sem):
    wid = lax.axis_index("s") * 2 + lax.axis_index("c")
    for s in range(N // SUB // NW):
        ch = s * NW + wid
        pltpu.sync_copy(sidx.at[pl.ds(ch * SUB, SUB)], sidx_v)
        pltpu.async_copy(src.at[sidx_v], rows_v, sem).wait()
        pltpu.sync_copy(rows_v, out.at[pl.ds(ch * SUB, SUB)])


def _mm_body(meta_ref, b0, b1, b2, b3, p0, p1, p2, p3, o_ref):
    j = pl.program_id(0)
    c = meta_ref[0, j]
    xs = (b0, b1, b2, b3)
    ws = (p0, p1, p2, p3)
    for i in range(4):
        @pl.when(c == i)
        def _(i=i):
            o_ref[...] = lax.dot_general(
                xs[i][...].astype(jnp.bfloat16), ws[i][...].astype(jnp.bfloat16),
                (((1,), (1,)), ((), ())),
                preferred_element_type=jnp.float32) * SCALE


def kernel(inp, table_0, table_1, table_2, table_3,
           proj_0, proj_1, proj_2, proj_3):
    inp_flat = inp.reshape(-1).astype(jnp.int32)
    meta, gidx, counts16, slot = _plan(inp_flat)

    bufs = pl.kernel(
        _gather_body,
        out_type=[jax.ShapeDtypeStruct((CAP, d), jnp.float32) for d in DSP],
        mesh=_mesh(),
        scratch_types=[
            pltpu.VMEM((SUB,), jnp.int32),
            pltpu.VMEM((SUB, DSP[0]), jnp.float32),
            pltpu.VMEM((SUB, DSP[1]), jnp.float32),
            pltpu.VMEM((SUB, DSP[2]), jnp.float32),
            pltpu.VMEM((SUB, DSP[3]), jnp.float32),
            pltpu.VMEM((16,), jnp.int32),
            pltpu.SemaphoreType.DMA,
        ],
    )(table_0, table_1, table_2, table_3, gidx, counts16)

    mm_out = pl.pallas_call(
        _mm_body,
        grid_spec=pltpu.PrefetchScalarGridSpec(
            num_scalar_prefetch=1,
            grid=(NT,),
            in_specs=[
                pl.BlockSpec((TILE, DSP[0]), lambda j, m: (m[1, j], 0)),
                pl.BlockSpec((TILE, DSP[1]), lambda j, m: (m[2, j], 0)),
                pl.BlockSpec((TILE, DSP[2]), lambda j, m: (m[3, j], 0)),
                pl.BlockSpec((TILE, DSP[3]), lambda j, m: (m[4, j], 0)),
                pl.BlockSpec((DP, DSP[0]), lambda j, m: (0, 0)),
                pl.BlockSpec((DP, DSP[1]), lambda j, m: (0, 0)),
                pl.BlockSpec((DP, DSP[2]), lambda j, m: (0, 0)),
                pl.BlockSpec((DP, DSP[3]), lambda j, m: (0, 0)),
            ],
            out_specs=pl.BlockSpec((TILE, DP), lambda j, m: (j, 0)),
        ),
        out_shape=jax.ShapeDtypeStruct((NT * TILE, DP), jnp.float32),
    )(meta, bufs[0], bufs[1], bufs[2], bufs[3],
      proj_0, proj_1, proj_2, proj_3)

    scattered = pl.kernel(
        _unsort_body,
        out_type=jax.ShapeDtypeStruct((N, DP), jnp.float32),
        mesh=_mesh(),
        scratch_types=[
            pltpu.VMEM((SUB, DP), jnp.float32),
            pltpu.VMEM((SUB,), jnp.int32),
            pltpu.SemaphoreType.DMA,
        ],
    )(mm_out, slot)

    return scattered.reshape(inp.shape + (DP,))


# speedup vs baseline: 1.1638x; 1.1638x over previous
import jax
import jax.numpy as jnp
from jax import lax
from jax.experimental import pallas as pl
from jax.experimental.pallas import tpu as pltpu
from jax.experimental.pallas import tpu_sc as plsc

N = 8192
DP = 1024
CUT = (0, 20000, 40000, 80000, 100000)
DS = (1024, 256, 64, 16)
DSP = (1024, 256, 64, 16)
SCALE = float(DP) ** 0.5
TILE = 256
SUB = 64
NT = 36
CPT = 4
CAP = 8192
NW = 32
MAXG = 4
MAXS = 5


def _mesh():
    return plsc.VectorSubcoreMesh(core_axis_name="c", subcore_axis_name="s")


def _plan(inp_flat):
    i32 = jnp.int32
    cid = ((inp_flat >= CUT[1]).astype(i32)
           + (inp_flat >= CUT[2]).astype(i32)
           + (inp_flat >= CUT[3]).astype(i32))
    perm = jnp.argsort(cid).astype(i32)
    n = jnp.stack([jnp.sum(cid == c) for c in range(4)]).astype(i32)
    b = jnp.concatenate([jnp.zeros((1,), i32), jnp.cumsum(n)[:-1].astype(i32)])
    t = (n + TILE - 1) // TILE
    T = jnp.concatenate([jnp.zeros((1,), i32), jnp.cumsum(t)[:-1].astype(i32)])
    ntt = jnp.sum(t)

    jj = jnp.arange(NT, dtype=i32)
    cid_t = ((jj >= T[1]).astype(i32) + (jj >= T[2]).astype(i32)
             + (jj >= T[3]).astype(i32))
    ks = [jnp.clip(jj - T[c], 0, jnp.maximum(t[c] - 1, 0)) for c in range(4)]
    meta = jnp.stack([cid_t] + ks).astype(i32)

    counts16 = jnp.concatenate(
        [t * CPT, (ntt * CPT)[None], jnp.zeros((3,), i32), T * CPT,
         jnp.zeros((4,), i32)]).astype(i32)

    r = jnp.arange(NT * TILE, dtype=i32)
    jr = r // TILE
    lr = r % TILE
    cid_r = ((jr >= T[1]).astype(i32) + (jr >= T[2]).astype(i32)
             + (jr >= T[3]).astype(i32))
    def sel(vals):
        x = jnp.where(cid_r == 1, vals[1], vals[0])
        x = jnp.where(cid_r == 2, vals[2], x)
        return jnp.where(cid_r == 3, vals[3], x)
    off = (jr - sel(T)) * TILE + lr
    valid = (jr < ntt) & (off < sel(n))
    pos = sel(b) + off
    dval = jnp.take(perm, jnp.clip(pos, 0, N - 1))
    tok = jnp.take(inp_flat, dval)
    cut_r = sel([jnp.full((), CUT[c], i32) for c in range(4)])
    gidx = jnp.where(valid, tok - cut_r, 0).astype(i32)
    dest = jnp.where(valid, dval, N).astype(i32)
    slot = jnp.zeros((N + 1,), i32).at[dest].set(r)[:N]
    return meta, gidx, counts16, slot


def _gather_body(t0, t1, t2, t3, g, cnts,
                 b0, b1, b2, b3, idx_v, r0, r1, r2, r3, cnt_v, sem):
    wid = lax.axis_index("s") * 2 + lax.axis_index("c")
    pltpu.sync_copy(cnts, cnt_v)
    cv = cnt_v[...]
    tabs = (t0, t1, t2, t3)
    bufs = (b0, b1, b2, b3)
    rvs = (r0, r1, r2, r3)
    for c in range(4):
        n_ch = cv[c]
        g_off = cv[8 + c]
        for s in range(MAXG):
            ch = s * NW + wid
            @pl.when(ch < n_ch)
            def _(c=c, ch=ch, g_off=g_off):
                pltpu.sync_copy(g.at[pl.ds((g_off + ch) * SUB, SUB)], idx_v)
                if c < 2:
                    pltpu.async_copy(tabs[c].at[idx_v], rvs[c], sem).wait()
                else:
                    descs = []
                    for k in range(SUB // 16):
                        xv = idx_v[pl.ds(k * 16, 16)]
                        for j in range(16):
                            descs.append(pltpu.async_copy(
                                tabs[c].at[xv[j]], rvs[c].at[k * 16 + j], sem))
                    for d in descs:
                        d.wait()
                pltpu.sync_copy(rvs[c], bufs[c].at[pl.ds(ch * SUB, SUB)])


def _unsort_body(src, sidx, out, rows_v, sidx_v, sem):
    wid = lax.axis_index("s") * 2 + lax.axis_index("c")
    for s in range(N // SUB // NW):
        ch = s * NW + wid
        pltpu.sync_copy(sidx.at[pl.ds(ch * SUB, SUB)], sidx_v)
        pltpu.async_copy(src.at[sidx_v], rows_v, sem).wait()
        pltpu.sync_copy(rows_v, out.at[pl.ds(ch * SUB, SUB)])


def _mm_body(meta_ref, b0, b1, b2, b3, p0, p1, p2, p3, o_ref):
    j = pl.program_id(0)
    c = meta_ref[0, j]
    xs = (b0, b1, b2, b3)
    ws = (p0, p1, p2, p3)
    for i in range(4):
        @pl.when(c == i)
        def _(i=i):
            o_ref[...] = lax.dot_general(
                xs[i][...].astype(jnp.bfloat16), ws[i][...].astype(jnp.bfloat16),
                (((1,), (1,)), ((), ())),
                preferred_element_type=jnp.float32) * SCALE


def kernel(inp, table_0, table_1, table_2, table_3,
           proj_0, proj_1, proj_2, proj_3):
    inp_flat = inp.reshape(-1).astype(jnp.int32)
    meta, gidx, counts16, slot = _plan(inp_flat)

    bufs = pl.kernel(
        _gather_body,
        out_type=[jax.ShapeDtypeStruct((CAP, d), jnp.float32) for d in DSP],
        mesh=_mesh(),
        scratch_types=[
            pltpu.VMEM((SUB,), jnp.int32),
            pltpu.VMEM((SUB, DSP[0]), jnp.float32),
            pltpu.VMEM((SUB, DSP[1]), jnp.float32),
            pltpu.VMEM((SUB, DSP[2]), jnp.float32),
            pltpu.VMEM((SUB, DSP[3]), jnp.float32),
            pltpu.VMEM((16,), jnp.int32),
            pltpu.SemaphoreType.DMA,
        ],
    )(table_0, table_1, table_2, table_3, gidx, counts16)

    mm_out = pl.pallas_call(
        _mm_body,
        grid_spec=pltpu.PrefetchScalarGridSpec(
            num_scalar_prefetch=1,
            grid=(NT,),
            in_specs=[
                pl.BlockSpec((TILE, DSP[0]), lambda j, m: (m[1, j], 0)),
                pl.BlockSpec((TILE, DSP[1]), lambda j, m: (m[2, j], 0)),
                pl.BlockSpec((TILE, DSP[2]), lambda j, m: (m[3, j], 0)),
                pl.BlockSpec((TILE, DSP[3]), lambda j, m: (m[4, j], 0)),
                pl.BlockSpec((DP, DSP[0]), lambda j, m: (0, 0)),
                pl.BlockSpec((DP, DSP[1]), lambda j, m: (0, 0)),
                pl.BlockSpec((DP, DSP[2]), lambda j, m: (0, 0)),
                pl.BlockSpec((DP, DSP[3]), lambda j, m: (0, 0)),
            ],
            out_specs=pl.BlockSpec((TILE, DP), lambda j, m: (j, 0)),
        ),
        out_shape=jax.ShapeDtypeStruct((NT * TILE, DP), jnp.float32),
    )(meta, bufs[0], bufs[1], bufs[2], bufs[3],
      proj_0, proj_1, proj_2, proj_3)

    scattered = pl.kernel(
        _unsort_body,
        out_type=jax.ShapeDtypeStruct((N, DP), jnp.float32),
        mesh=_mesh(),
        scratch_types=[
            pltpu.VMEM((SUB, DP), jnp.float32),
            pltpu.VMEM((SUB,), jnp.int32),
            pltpu.SemaphoreType.DMA,
        ],
    )(mm_out, slot)

    return scattered.reshape(inp.shape + (DP,))
